# baseline (speedup 1.0000x reference)
import functools

import jax
import jax.numpy as jnp
from jax import lax
from jax.experimental import pallas as pl
from jax.experimental.pallas import tpu as pltpu
from jax.experimental.pallas import tpu_sc as plsc

_N_LOCATIONS = 16384
_DIM = 64
_SIDE_BITS = 7
_MAX_DEPTH = 7
_BATCH = 4096
_N_LEAVES = 4 ** _MAX_DEPTH


def _query_index(loc):
    x = loc & (2 ** _SIDE_BITS - 1)
    y = loc >> _SIDE_BITS
    c7 = 2 * (y & 1) + (x & 1)
    q6 = jnp.zeros_like(loc)
    for u in range(1, _SIDE_BITS):
        q6 = q6 + ((2 * ((y >> u) & 1) + ((x >> u) & 1)) << (2 * (_SIDE_BITS - 1 - u)))
    return ((c7 >> 1) << 13) + 2 * q6 + (c7 & 1)


def _expand_kernel(loc_ref, table_ref, W_ref, b_ref, W6_ref, b6_ref,
                   out_ref, idx_ref, s6_ref):
    j = pl.program_id(0)
    qrows = _N_LEAVES // 8

    @pl.when(j == 0)
    def _chain():
        idx_ref[:, :] = _query_index(loc_ref[:, :])
        s = table_ref[0:1, :]
        for d in range(_MAX_DEPTH - 1):
            y = jnp.dot(s, W_ref[d], preferred_element_type=jnp.float32)
            y = y + b_ref[d][None, :]
            s = jnp.concatenate(
                [y[:, _DIM * c:_DIM * (c + 1)] for c in range(4)], axis=0)
        s6_ref[0] = s[0:qrows]
        s6_ref[1] = s[qrows:2 * qrows]

    s = s6_ref[j % 2]
    y = jnp.dot(s, W6_ref[0], preferred_element_type=jnp.float32)
    out_ref[:, :] = y + b6_ref[0, 0][None, :]


def _expand_states(location, table, W, b):
    qrows = _N_LEAVES // 8
    return pl.pallas_call(
        _expand_kernel,
        grid=(4,),
        in_specs=[
            pl.BlockSpec((_BATCH // 128, 128), lambda j: (0, 0)),
            pl.BlockSpec((1 + 3, _DIM), lambda j: (0, 0)),
            pl.BlockSpec((_MAX_DEPTH, _DIM, 4 * _DIM), lambda j: (0, 0, 0)),
            pl.BlockSpec((_MAX_DEPTH, 4 * _DIM), lambda j: (0, 0)),
            pl.BlockSpec((1, _DIM, 2 * _DIM),
                         lambda j: (_MAX_DEPTH - 1, 0, j // 2)),
            pl.BlockSpec((1, 1, 2 * _DIM),
                         lambda j: (_MAX_DEPTH - 1, 0, j // 2)),
        ],
        out_specs=(
            pl.BlockSpec((qrows, 2 * _DIM), lambda j: (j, 0)),
            pl.BlockSpec((_BATCH // 128, 128), lambda j: (0, 0)),
        ),
        out_shape=(
            jax.ShapeDtypeStruct((_N_LEAVES // 2, 2 * _DIM), jnp.float32),
            jax.ShapeDtypeStruct((_BATCH // 128, 128), jnp.int32),
        ),
        scratch_shapes=[pltpu.VMEM((2, _N_LEAVES // 8, _DIM), jnp.float32)],
    )(location.reshape(_BATCH // 128, 128), table, W, b, W,
      b.reshape(_MAX_DEPTH, 1, 4 * _DIM))


_SC_INFO = plsc.get_sparse_core_info()
_NC = _SC_INFO.num_cores
_NW = _NC * _SC_INFO.num_subcores
_B_PER_W = _BATCH // _NW


@functools.partial(
    pl.kernel,
    mesh=plsc.VectorSubcoreMesh(core_axis_name="c", subcore_axis_name="s"),
    out_type=jax.ShapeDtypeStruct((_BATCH, _DIM), jnp.float32),
    scratch_types=[
        pltpu.VMEM((_B_PER_W,), jnp.int32),
        pltpu.VMEM((_B_PER_W, _DIM), jnp.float32),
        pltpu.SemaphoreType.DMA,
    ],
    compiler_params=pltpu.CompilerParams(use_tc_tiling_on_sc=False),
)
def _sc_gather(idx_hbm, leaf_hbm, out_hbm, idx_v, rows_v, sem):
    wid = lax.axis_index("s") * _NC + lax.axis_index("c")
    pltpu.sync_copy(idx_hbm.at[wid], idx_v)
    pltpu.async_copy(leaf_hbm.at[idx_v], rows_v, sem).wait()
    pltpu.sync_copy(rows_v, out_hbm.at[pl.ds(wid * _B_PER_W, _B_PER_W)])


def kernel(location, table, W, b):
    pairs, idx = _expand_states(location, table, W, b)
    leaf = pairs.reshape(_N_LEAVES, _DIM)
    return _sc_gather(idx, leaf)


# speedup vs baseline: 1.0769x; 1.0769x over previous
import functools

import jax
import jax.numpy as jnp
from jax import lax
from jax.experimental import pallas as pl
from jax.experimental.pallas import tpu as pltpu
from jax.experimental.pallas import tpu_sc as plsc

_N_LOCATIONS = 16384
_DIM = 64
_SIDE_BITS = 7
_MAX_DEPTH = 7
_BATCH = 4096
_N_LEAVES = 4 ** _MAX_DEPTH


def _query_index(loc):
    x = loc & (2 ** _SIDE_BITS - 1)
    y = loc >> _SIDE_BITS
    c7 = 2 * (y & 1) + (x & 1)
    q6 = jnp.zeros_like(loc)
    for u in range(1, _SIDE_BITS):
        q6 = q6 + ((2 * ((y >> u) & 1) + ((x >> u) & 1)) << (2 * (_SIDE_BITS - 1 - u)))
    return ((c7 >> 1) << 13) + 2 * q6 + (c7 & 1)


def _expand_kernel(loc_ref, table_ref, W_ref, b_ref, out_ref, idx_ref):
    idx_ref[:, :] = _query_index(loc_ref[:, :])
    s = table_ref[0:1, :]
    for d in range(_MAX_DEPTH - 1):
        y = jnp.dot(s, W_ref[d], preferred_element_type=jnp.float32)
        y = y + b_ref[d][None, :]
        s = jnp.concatenate(
            [y[:, _DIM * c:_DIM * (c + 1)] for c in range(4)], axis=0)
    y = jnp.dot(s, W_ref[_MAX_DEPTH - 1], preferred_element_type=jnp.float32)
    y = y + b_ref[_MAX_DEPTH - 1][None, :]
    half_rows = _N_LEAVES // 4
    out_ref[0:half_rows, :] = y[:, 0:2 * _DIM]
    out_ref[half_rows:2 * half_rows, :] = y[:, 2 * _DIM:4 * _DIM]


def _expand_states(location, table, W, b):
    return pl.pallas_call(
        _expand_kernel,
        out_shape=(
            jax.ShapeDtypeStruct((_N_LEAVES // 2, 2 * _DIM), jnp.float32),
            jax.ShapeDtypeStruct((_BATCH // 128, 128), jnp.int32),
        ),
    )(location.reshape(_BATCH // 128, 128), table, W, b)


_SC_INFO = plsc.get_sparse_core_info()
_NC = _SC_INFO.num_cores
_NW = _NC * _SC_INFO.num_subcores
_B_PER_W = _BATCH // _NW


@functools.partial(
    pl.kernel,
    mesh=plsc.VectorSubcoreMesh(core_axis_name="c", subcore_axis_name="s"),
    out_type=jax.ShapeDtypeStruct((_BATCH, _DIM), jnp.float32),
    scratch_types=[
        pltpu.VMEM((_B_PER_W,), jnp.int32),
        pltpu.VMEM((_B_PER_W, _DIM), jnp.float32),
        pltpu.SemaphoreType.DMA,
    ],
    compiler_params=pltpu.CompilerParams(use_tc_tiling_on_sc=False),
)
def _sc_gather(idx_hbm, leaf_hbm, out_hbm, idx_v, rows_v, sem):
    wid = lax.axis_index("s") * _NC + lax.axis_index("c")
    pltpu.sync_copy(idx_hbm.at[wid], idx_v)
    pltpu.async_copy(leaf_hbm.at[idx_v], rows_v, sem).wait()
    pltpu.sync_copy(rows_v, out_hbm.at[pl.ds(wid * _B_PER_W, _B_PER_W)])


def kernel(location, table, W, b):
    pairs, idx = _expand_states(location, table, W, b)
    leaf = pairs.reshape(_N_LEAVES, _DIM)
    return _sc_gather(idx, leaf)
